# baseline (speedup 1.0000x reference)
import functools

import jax
import jax.numpy as jnp
from jax import lax
from jax.experimental import pallas as pl
from jax.experimental.pallas import tpu as pltpu
from jax.experimental.pallas import tpu_sc as plsc

N_USERS = 1000000
N_ITEMS = 1000000
EMB_DIM = 16
HIDDEN_DIM = 128
BATCH = 16384

NC, NS = 2, 16
NW = NC * NS
BPW = BATCH // NW
CHUNK = 128
NCH = BPW // CHUNK

MLP_BLK = 2048


def _gather_body(user_hbm, item_hbm, u2_hbm, i2_hbm, ue_hbm, ie_hbm,
                 idx_u, idx_i, rows_u, rows_i, sem):
    wid = lax.axis_index("s") * NC + lax.axis_index("c")
    base = wid * BPW
    pltpu.sync_copy(u2_hbm.at[pl.ds(wid * NCH, NCH)], idx_u)
    pltpu.sync_copy(i2_hbm.at[pl.ds(wid * NCH, NCH)], idx_i)
    copies = []
    for j in range(NCH):
        copies.append(pltpu.async_copy(
            user_hbm.at[idx_u.at[j]], rows_u.at[pl.ds(j * CHUNK, CHUNK)], sem))
        copies.append(pltpu.async_copy(
            item_hbm.at[idx_i.at[j]], rows_i.at[pl.ds(j * CHUNK, CHUNK)], sem))
    for c in copies:
        c.wait()
    pltpu.sync_copy(rows_u, ue_hbm.at[pl.ds(base, BPW)])
    pltpu.sync_copy(rows_i, ie_hbm.at[pl.ds(base, BPW)])


def _mlp_body(ue_ref, ie_ref, w1u_ref, w1i_ref, b1_ref, w2_ref, b2_ref, out_ref):
    h = (jnp.dot(ue_ref[...], w1u_ref[...], preferred_element_type=jnp.float32)
         + jnp.dot(ie_ref[...], w1i_ref[...], preferred_element_type=jnp.float32)
         + b1_ref[...])
    h = jnp.maximum(h, 0.0)
    out_ref[...] = jnp.sum(h * w2_ref[...], axis=1) + b2_ref[0, 0]


def _gather_call(user_emb, item_emb, u2, i2):
    return pl.kernel(
        _gather_body,
        mesh=plsc.VectorSubcoreMesh(core_axis_name="c", subcore_axis_name="s"),
        compiler_params=pltpu.CompilerParams(use_tc_tiling_on_sc=False),
        out_type=[jax.ShapeDtypeStruct((BATCH, EMB_DIM), jnp.float32),
                  jax.ShapeDtypeStruct((BATCH, EMB_DIM), jnp.float32)],
        scratch_types=[
            pltpu.VMEM((NCH, CHUNK), jnp.int32),
            pltpu.VMEM((NCH, CHUNK), jnp.int32),
            pltpu.VMEM((BPW, EMB_DIM), jnp.float32),
            pltpu.VMEM((BPW, EMB_DIM), jnp.float32),
            pltpu.SemaphoreType.DMA,
        ],
    )(user_emb, item_emb, u2, i2)


def _mlp_call(ue, ie, w1u, w1i, b1r, w2r, b2r):
    return pl.pallas_call(
        _mlp_body,
        grid=(BATCH // MLP_BLK,),
        in_specs=[
            pl.BlockSpec((MLP_BLK, EMB_DIM), lambda j: (j, 0)),
            pl.BlockSpec((MLP_BLK, EMB_DIM), lambda j: (j, 0)),
            pl.BlockSpec((EMB_DIM, HIDDEN_DIM), lambda j: (0, 0)),
            pl.BlockSpec((EMB_DIM, HIDDEN_DIM), lambda j: (0, 0)),
            pl.BlockSpec((1, HIDDEN_DIM), lambda j: (0, 0)),
            pl.BlockSpec((1, HIDDEN_DIM), lambda j: (0, 0)),
            pl.BlockSpec((1, 1), lambda j: (0, 0)),
        ],
        out_specs=pl.BlockSpec((MLP_BLK,), lambda j: (j,)),
        out_shape=jax.ShapeDtypeStruct((BATCH,), jnp.float32),
    )(ue, ie, w1u, w1i, b1r, w2r, b2r)


def kernel(u, i, user_emb, item_emb, W1, b1, W2, b2):
    u2 = u.astype(jnp.int32).reshape(BATCH // CHUNK, CHUNK)
    i2 = i.astype(jnp.int32).reshape(BATCH // CHUNK, CHUNK)
    ue, ie = _gather_call(user_emb, item_emb, u2, i2)
    w1u = W1[:EMB_DIM]
    w1i = W1[EMB_DIM:]
    b1r = b1.reshape(1, HIDDEN_DIM)
    w2r = W2.reshape(1, HIDDEN_DIM)
    b2r = b2.reshape(1, 1)
    return _mlp_call(ue, ie, w1u, w1i, b1r, w2r, b2r)
